# Initial kernel scaffold; baseline (speedup 1.0000x reference)
import functools

import jax
import jax.numpy as jnp
from jax import lax
from jax.experimental import pallas as pl
from jax.experimental.pallas import tpu as pltpu
from jax.experimental.pallas import tpu_sc as plsc

_B, _N, _K, _CIN, _COUT = 2, 4096, 16, 128, 128
_D = 2 * _COUT
_CP = 3 + _CIN
_EPS = 1e-5

_NC, _NS = 2, 16
_NW = _NC * _NS
_PTS = _B * _N
_PW = _PTS // _NW
_P = 8
_NCHUNK = _PW // _P

_BLKA = 512


def _mm_body(p_ref, w_ref, nbr_ref, ctr_ref):
    r = jnp.dot(p_ref[...], w_ref[...], preferred_element_type=jnp.float32)
    nbr_ref[...] = r[:, :_D]
    ctr_ref[...] = r[:, _D:]


def _tables(p_all, w_comb):
    grid = _PTS // _BLKA
    return pl.pallas_call(
        _mm_body,
        grid=(grid,),
        in_specs=[
            pl.BlockSpec((_BLKA, _CP), lambda i: (i, 0)),
            pl.BlockSpec((_CP, 2 * _D), lambda i: (0, 0)),
        ],
        out_specs=[
            pl.BlockSpec((_BLKA, _D), lambda i: (i, 0)),
            pl.BlockSpec((_BLKA, _D), lambda i: (i, 0)),
        ],
        out_shape=[
            jax.ShapeDtypeStruct((_PTS, _D), jnp.float32),
            jax.ShapeDtypeStruct((_PTS, _D), jnp.float32),
        ],
    )(p_all, w_comb)


def _sc_body(nbr_hbm, idx_hbm, ctr_hbm, out_hbm, idx_v, rows_v, ctr_v, out_v,
             sem):
    wid = lax.axis_index("s") * _NC + lax.axis_index("c")
    pbase0 = wid * _PW

    def chunk_body(g, carry):
        pbase = pbase0 + g * _P
        pltpu.sync_copy(idx_hbm.at[pl.ds(pbase * _K, _P * _K)], idx_v)
        pltpu.async_copy(nbr_hbm.at[idx_v], rows_v, sem).wait()
        pltpu.sync_copy(ctr_hbm.at[pl.ds(pbase, _P)], ctr_v)

        def point_body(p, c2):
            def col_body(c, c3):
                sl = pl.ds(c * 16, 16)
                acc = rows_v[p * _K, sl]
                for r in range(1, _K):
                    acc = jnp.maximum(acc, rows_v[p * _K + r, sl])
                out_v[p, sl] = jnp.maximum(acc + ctr_v[p, sl], 0.0)
                return c3

            return lax.fori_loop(0, _D // 16, col_body, c2)

        lax.fori_loop(0, _P, point_body, 0)
        pltpu.sync_copy(out_v, out_hbm.at[pl.ds(pbase, _P)])
        return carry

    lax.fori_loop(0, _NCHUNK, chunk_body, 0)


def _gather_max(t_nbr, idx_flat, t_ctr):
    mesh = plsc.VectorSubcoreMesh(core_axis_name="c", subcore_axis_name="s")
    fn = pl.kernel(
        _sc_body,
        out_type=jax.ShapeDtypeStruct((_PTS, _D), jnp.float32),
        mesh=mesh,
        scratch_types=[
            pltpu.VMEM((_P * _K,), jnp.int32),
            pltpu.VMEM((_P * _K, _D), jnp.float32),
            pltpu.VMEM((_P, _D), jnp.float32),
            pltpu.VMEM((_P, _D), jnp.float32),
            pltpu.SemaphoreType.DMA,
        ],
    )
    return fn(t_nbr, idx_flat, t_ctr)


def _tr_body(i_ref, o_ref):
    o_ref[0] = i_ref[0].T


def _to_bcn(out_t):
    blk = 512
    return pl.pallas_call(
        _tr_body,
        grid=(_B, _N // blk),
        in_specs=[pl.BlockSpec((1, blk, _D), lambda b, j: (b, j, 0))],
        out_specs=pl.BlockSpec((1, _D, blk), lambda b, j: (b, 0, j)),
        out_shape=jax.ShapeDtypeStruct((_B, _D, _N), jnp.float32),
    )(out_t.reshape(_B, _N, _D))


@jax.jit
def kernel(fea, x, idx, W1, g1, b1, W2, g2, b2):
    inv = 1.0 / jnp.sqrt(1.0 + _EPS)
    s1 = (g1 * inv)[:, None]
    s2 = (g2 * inv)[:, None]
    w1n = (W1[:, 3:] * s1).T
    w1c = ((W1[:, :3] - W1[:, 3:]) * s1).T
    w2n = (W2[:, _CIN:] * s2).T
    w2c = ((W2[:, :_CIN] - W2[:, _CIN:]) * s2).T

    z31 = jnp.zeros((3, _COUT), jnp.float32)
    z131 = jnp.zeros((_CIN, _COUT), jnp.float32)
    w_comb = jnp.concatenate(
        [
            jnp.concatenate([w1n, z31, w1c, z31], axis=1),
            jnp.concatenate([z131, w2n, z131, w2c], axis=1),
        ],
        axis=0,
    )

    p_all = jnp.concatenate(
        [jnp.swapaxes(x, 1, 2), jnp.swapaxes(fea, 1, 2)], axis=2
    ).reshape(_PTS, _CP)

    t_nbr, t_ctr = _tables(p_all, w_comb)
    t_ctr = t_ctr + jnp.concatenate([b1, b2])[None, :]

    idx_flat = (
        jnp.swapaxes(idx, 1, 2) + (jnp.arange(_B, dtype=jnp.int32) * _N)[:, None, None]
    ).reshape(_PTS * _K)

    out_t = _gather_max(t_nbr, idx_flat, t_ctr)
    return _to_bcn(out_t)


# speedup vs baseline: 15.7427x; 15.7427x over previous
import functools

import jax
import jax.numpy as jnp
from jax import lax
from jax.experimental import pallas as pl
from jax.experimental.pallas import tpu as pltpu
from jax.experimental.pallas import tpu_sc as plsc

_B, _N, _K, _CIN, _COUT = 2, 4096, 16, 128, 128
_D = 2 * _COUT
_CP = 3 + _CIN + 1
_EPS = 1e-5

_NC, _NS = 2, 16
_NW = _NC * _NS
_PTS = _B * _N
_PW = _PTS // _NW
_P = 8
_NCHUNK = _PW // _P

_BLKA = 512


def _mm_body(p_ref, w_ref, nbr_ref, ctr_ref):
    r = jnp.dot(p_ref[...], w_ref[...], preferred_element_type=jnp.float32)
    nbr_ref[...] = r[:, :_D]
    ctr_ref[...] = r[:, _D:]


def _tables(p_all, w_comb):
    grid = _PTS // _BLKA
    return pl.pallas_call(
        _mm_body,
        grid=(grid,),
        in_specs=[
            pl.BlockSpec((_BLKA, _CP), lambda i: (i, 0)),
            pl.BlockSpec((_CP, 2 * _D), lambda i: (0, 0)),
        ],
        out_specs=[
            pl.BlockSpec((_BLKA, _D), lambda i: (i, 0)),
            pl.BlockSpec((_BLKA, _D), lambda i: (i, 0)),
        ],
        out_shape=[
            jax.ShapeDtypeStruct((_PTS, _D), jnp.float32),
            jax.ShapeDtypeStruct((_PTS, _D), jnp.float32),
        ],
    )(p_all, w_comb)


def _sc_body(nbr_hbm, idx_hbm, ctr_hbm, out_hbm, idx_v, rows_v, ctr_v, out_v,
             sem):
    wid = lax.axis_index("s") * _NC + lax.axis_index("c")
    pbase0 = wid * _PW

    def chunk_body(g, carry):
        pbase = pbase0 + g * _P
        pltpu.sync_copy(idx_hbm.at[pl.ds(pbase * _K, _P * _K)], idx_v)
        pltpu.async_copy(nbr_hbm.at[idx_v], rows_v, sem).wait()
        pltpu.sync_copy(ctr_hbm.at[pl.ds(pbase, _P)], ctr_v)

        def point_body(p, c2):
            def col_body(c, c3):
                sl = pl.ds(c * 16, 16)
                acc = rows_v[p * _K, sl]
                for r in range(1, _K):
                    acc = jnp.maximum(acc, rows_v[p * _K + r, sl])
                out_v[p, sl] = jnp.maximum(acc + ctr_v[p, sl], 0.0)
                return c3

            return lax.fori_loop(0, _D // 16, col_body, c2)

        lax.fori_loop(0, _P, point_body, 0)
        pltpu.sync_copy(out_v, out_hbm.at[pl.ds(pbase, _P)])
        return carry

    lax.fori_loop(0, _NCHUNK, chunk_body, 0)


def _gather_max(t_nbr, idx_flat, t_ctr):
    mesh = plsc.VectorSubcoreMesh(
        core_axis_name="c", subcore_axis_name="s", num_cores=_NC,
        num_subcores=_NS,
    )
    fn = pl.kernel(
        _sc_body,
        out_type=jax.ShapeDtypeStruct((_PTS, _D), jnp.float32),
        mesh=mesh,
        scratch_types=[
            pltpu.VMEM((_P * _K,), jnp.int32),
            pltpu.VMEM((_P * _K, _D), jnp.float32),
            pltpu.VMEM((_P, _D), jnp.float32),
            pltpu.VMEM((_P, _D), jnp.float32),
            pltpu.SemaphoreType.DMA,
        ],
    )
    return fn(t_nbr, idx_flat, t_ctr)


def _tr_body(i_ref, o_ref):
    o_ref[0] = i_ref[0].T


def _to_bcn(out_t):
    blk = 512
    return pl.pallas_call(
        _tr_body,
        grid=(_B, _N // blk),
        in_specs=[pl.BlockSpec((1, blk, _D), lambda b, j: (b, j, 0))],
        out_specs=pl.BlockSpec((1, _D, blk), lambda b, j: (b, 0, j)),
        out_shape=jax.ShapeDtypeStruct((_B, _D, _N), jnp.float32),
    )(out_t.reshape(_B, _N, _D))


@jax.jit
def kernel(fea, x, idx, W1, g1, b1, W2, g2, b2):
    inv = 1.0 / jnp.sqrt(1.0 + _EPS)
    s1 = (g1 * inv)[:, None]
    s2 = (g2 * inv)[:, None]
    w1n = (W1[:, 3:] * s1).T
    w1c = ((W1[:, :3] - W1[:, 3:]) * s1).T
    w2n = (W2[:, _CIN:] * s2).T
    w2c = ((W2[:, :_CIN] - W2[:, _CIN:]) * s2).T

    z31 = jnp.zeros((3, _COUT), jnp.float32)
    z131 = jnp.zeros((_CIN, _COUT), jnp.float32)
    zD = jnp.zeros((_D,), jnp.float32)
    w_comb = jnp.concatenate(
        [
            jnp.concatenate([w1n, z31, w1c, z31], axis=1),
            jnp.concatenate([z131, w2n, z131, w2c], axis=1),
            jnp.concatenate([zD, b1, b2])[None, :],
        ],
        axis=0,
    )

    p_all = jnp.concatenate(
        [
            jnp.swapaxes(x, 1, 2),
            jnp.swapaxes(fea, 1, 2),
            jnp.ones((_B, _N, 1), jnp.float32),
        ],
        axis=2,
    ).reshape(_PTS, _CP)

    t_nbr, t_ctr = _tables(p_all, w_comb)

    idx_flat = (
        jnp.swapaxes(idx, 1, 2) + (jnp.arange(_B, dtype=jnp.int32) * _N)[:, None, None]
    ).reshape(_PTS * _K)

    out_t = _gather_max(t_nbr, idx_flat, t_ctr)
    return _to_bcn(out_t)


# speedup vs baseline: 27.0143x; 1.7160x over previous
import functools

import jax
import jax.numpy as jnp
from jax import lax
from jax.experimental import pallas as pl
from jax.experimental.pallas import tpu as pltpu
from jax.experimental.pallas import tpu_sc as plsc

_B, _N, _K, _CIN, _COUT = 2, 4096, 16, 128, 128
_D = 2 * _COUT
_CP = 3 + _CIN + 1
_EPS = 1e-5

_NC, _NS = 2, 16
_NW = _NC * _NS
_PTS = _B * _N
_PW = _PTS // _NW
_P = 8
_NCHUNK = _PW // _P

_BLKA = 512


def _mm_body(p_ref, w_ref, nbr_ref, ctr_ref):
    r = jnp.dot(p_ref[...], w_ref[...], preferred_element_type=jnp.float32)
    nbr_ref[...] = r[:, :_D]
    ctr_ref[...] = r[:, _D:]


def _tables(p_all, w_comb):
    grid = _PTS // _BLKA
    return pl.pallas_call(
        _mm_body,
        grid=(grid,),
        in_specs=[
            pl.BlockSpec((_BLKA, _CP), lambda i: (i, 0)),
            pl.BlockSpec((_CP, 2 * _D), lambda i: (0, 0)),
        ],
        out_specs=[
            pl.BlockSpec((_BLKA, _D), lambda i: (i, 0)),
            pl.BlockSpec((_BLKA, _D), lambda i: (i, 0)),
        ],
        out_shape=[
            jax.ShapeDtypeStruct((_PTS, _D), jnp.float32),
            jax.ShapeDtypeStruct((_PTS, _D), jnp.float32),
        ],
    )(p_all, w_comb)


def _sc_body(nbr_hbm, idx_hbm, ctr_hbm, out_hbm, idx_all,
             rows0, rows1, ctr0, ctr1, out0, out1,
             gsem0, gsem1, csem0, csem1, osem0, osem1):
    wid = lax.axis_index("s") * _NC + lax.axis_index("c")
    pbase0 = wid * _PW
    rows = (rows0, rows1)
    ctr = (ctr0, ctr1)
    out = (out0, out1)
    gsem = (gsem0, gsem1)
    csem = (csem0, csem1)
    osem = (osem0, osem1)

    pltpu.sync_copy(idx_hbm.at[wid], idx_all)

    def fetch(g, buf):
        pbase = pbase0 + g * _P
        pltpu.async_copy(nbr_hbm.at[idx_all.at[g]], rows[buf], gsem[buf])
        pltpu.async_copy(ctr_hbm.at[pl.ds(pbase, _P)], ctr[buf], csem[buf])

    fetch(0, 0)

    def chunk_body(g, cur):
        nxt = 1 - cur

        @pl.when(g + 1 < _NCHUNK)
        def _():
            fetch(g + 1, nxt)

        pltpu.make_async_copy(nbr_hbm.at[idx_all.at[g]], rows[cur],
                              gsem[cur]).wait()
        pltpu.make_async_copy(ctr_hbm.at[pl.ds(0, _P)], ctr[cur],
                              csem[cur]).wait()

        @pl.when(g >= 2)
        def _():
            pltpu.make_async_copy(out[cur], out_hbm.at[pl.ds(0, _P)],
                                  osem[cur]).wait()

        def point_body(p, c2):
            def col_body(c, c3):
                sl = pl.ds(c * 16, 16)
                acc = rows[cur][p * _K, sl]
                for r in range(1, _K):
                    acc = jnp.maximum(acc, rows[cur][p * _K + r, sl])
                out[cur][p, sl] = jnp.maximum(acc + ctr[cur][p, sl], 0.0)
                return c3

            return lax.fori_loop(0, _D // 16, col_body, c2)

        lax.fori_loop(0, _P, point_body, 0)
        pltpu.async_copy(out[cur], out_hbm.at[pl.ds(pbase0 + g * _P, _P)],
                         osem[cur])
        return nxt

    def pair_body(go, carry):
        for b in range(2):
            chunk_body(2 * go + b, b)
        return carry

    lax.fori_loop(0, _NCHUNK // 2, pair_body, 0)

    for b in range(2):
        pltpu.make_async_copy(out[b], out_hbm.at[pl.ds(0, _P)],
                              osem[b]).wait()


def _gather_max(t_nbr, idx_by_worker, t_ctr):
    mesh = plsc.VectorSubcoreMesh(
        core_axis_name="c", subcore_axis_name="s", num_cores=_NC,
        num_subcores=_NS,
    )
    fn = pl.kernel(
        _sc_body,
        out_type=jax.ShapeDtypeStruct((_PTS, _D), jnp.float32),
        mesh=mesh,
        scratch_types=[
            pltpu.VMEM((_NCHUNK, _P * _K), jnp.int32),
            pltpu.VMEM((_P * _K, _D), jnp.float32),
            pltpu.VMEM((_P * _K, _D), jnp.float32),
            pltpu.VMEM((_P, _D), jnp.float32),
            pltpu.VMEM((_P, _D), jnp.float32),
            pltpu.VMEM((_P, _D), jnp.float32),
            pltpu.VMEM((_P, _D), jnp.float32),
            pltpu.SemaphoreType.DMA,
            pltpu.SemaphoreType.DMA,
            pltpu.SemaphoreType.DMA,
            pltpu.SemaphoreType.DMA,
            pltpu.SemaphoreType.DMA,
            pltpu.SemaphoreType.DMA,
        ],
    )
    return fn(t_nbr, idx_by_worker, t_ctr)


def _tr_body(i_ref, o_ref):
    o_ref[0] = i_ref[0].T


def _to_bcn(out_t):
    blk = 512
    return pl.pallas_call(
        _tr_body,
        grid=(_B, _N // blk),
        in_specs=[pl.BlockSpec((1, blk, _D), lambda b, j: (b, j, 0))],
        out_specs=pl.BlockSpec((1, _D, blk), lambda b, j: (b, 0, j)),
        out_shape=jax.ShapeDtypeStruct((_B, _D, _N), jnp.float32),
    )(out_t.reshape(_B, _N, _D))


@jax.jit
def kernel(fea, x, idx, W1, g1, b1, W2, g2, b2):
    inv = 1.0 / jnp.sqrt(1.0 + _EPS)
    s1 = (g1 * inv)[:, None]
    s2 = (g2 * inv)[:, None]
    w1n = (W1[:, 3:] * s1).T
    w1c = ((W1[:, :3] - W1[:, 3:]) * s1).T
    w2n = (W2[:, _CIN:] * s2).T
    w2c = ((W2[:, :_CIN] - W2[:, _CIN:]) * s2).T

    z31 = jnp.zeros((3, _COUT), jnp.float32)
    z131 = jnp.zeros((_CIN, _COUT), jnp.float32)
    zD = jnp.zeros((_D,), jnp.float32)
    w_comb = jnp.concatenate(
        [
            jnp.concatenate([w1n, z31, w1c, z31], axis=1),
            jnp.concatenate([z131, w2n, z131, w2c], axis=1),
            jnp.concatenate([zD, b1, b2])[None, :],
        ],
        axis=0,
    )

    p_all = jnp.concatenate(
        [
            jnp.swapaxes(x, 1, 2),
            jnp.swapaxes(fea, 1, 2),
            jnp.ones((_B, _N, 1), jnp.float32),
        ],
        axis=2,
    ).reshape(_PTS, _CP)

    t_nbr, t_ctr = _tables(p_all, w_comb)

    idx_by_worker = (
        jnp.swapaxes(idx, 1, 2) + (jnp.arange(_B, dtype=jnp.int32) * _N)[:, None, None]
    ).reshape(_NW, _NCHUNK, _P * _K)

    out_t = _gather_max(t_nbr, idx_by_worker, t_ctr)
    return _to_bcn(out_t)
